# baseline (speedup 1.0000x reference)
import functools

import jax
import jax.numpy as jnp
from jax import lax
from jax.experimental import pallas as pl
from jax.experimental.pallas import tpu as pltpu
from jax.experimental.pallas import tpu_sc as plsc

_BATCH = 16384
_HIST = 50
_HIDDEN = 32
_B = _BATCH * _HIST
_NC, _NS = 2, 16
_NW = _NC * _NS
_BT = _BATCH // 128
_BT_PER_W = _BT // _NW
_G = 5
_NGRP = _HIST // _G

_mesh = plsc.VectorSubcoreMesh(core_axis_name="c", subcore_axis_name="s")


@functools.partial(
    pl.kernel,
    mesh=_mesh,
    out_type=jax.ShapeDtypeStruct((_BATCH, 2048), jnp.float32),
    compiler_params=pltpu.CompilerParams(use_tc_tiling_on_sc=False),
    scratch_types=[
        pltpu.VMEM((_HIST, 128), jnp.int32),
        pltpu.VMEM((_G * 128, _HIDDEN), jnp.float32),
        pltpu.VMEM((_G * 128, _HIDDEN), jnp.float32),
        pltpu.VMEM((_G * 128, _HIDDEN), jnp.float32),
        pltpu.SemaphoreType.DMA,
        pltpu.SemaphoreType.DMA,
        pltpu.SemaphoreType.DMA,
        pltpu.SemaphoreType.DMA,
        pltpu.SemaphoreType.DMA,
        pltpu.SemaphoreType.DMA,
    ],
)
def _embed(actions_hbm, table_hbm, out_hbm, idx_slab, slot0, slot1, slot2,
           gs0, gs1, gs2, os0, os1, os2):
    wid = lax.axis_index("s") * _NC + lax.axis_index("c")
    slots = (slot0, slot1, slot2)
    gsems = (gs0, gs1, gs2)
    osems = (os0, os1, os2)

    def fire(g, btg, s):
        for j in range(_G):
            h = g * _G + j
            pltpu.async_copy(
                table_hbm.at[idx_slab.at[h]],
                slots[s].at[pl.ds(j * 128, 128)],
                gsems[s],
            )

    def drain_gathers(s):
        pltpu.make_async_copy(
            table_hbm.at[pl.ds(0, _G * 128)], slots[s], gsems[s]
        ).wait()

    def store(g, btg, s):
        for j in range(_G):
            h = g * _G + j
            pltpu.async_copy(
                slots[s].at[pl.ds(j * 128, 128)],
                out_hbm.at[pl.ds(btg * 128, 128), pl.ds(h * _HIDDEN, _HIDDEN)],
                osems[s],
            )

    def wait_store(s):
        pltpu.make_async_copy(
            slots[s],
            out_hbm.at[pl.ds(0, _G * 128), pl.ds(0, _HIDDEN)],
            osems[s],
        ).wait()

    def btile_body(bt, carry):
        btg = wid * _BT_PER_W + bt
        pltpu.sync_copy(
            actions_hbm.at[pl.ds(0, _HIST), pl.ds(btg * 128, 128)], idx_slab)

        fire(0, btg, 0)
        fire(1, btg, 1)
        for g in range(_NGRP):
            s = g % 3
            drain_gathers(s)
            if g + 2 < _NGRP:
                ns = (g + 2) % 3
                if g >= 1:
                    wait_store(ns)
                fire(g + 2, btg, ns)
            store(g, btg, s)
        wait_store(0)
        wait_store(1)
        wait_store(2)
        return carry

    lax.fori_loop(0, _BT_PER_W, btile_body, 0)


def kernel(actions, embedding_table):
    actions_t = jnp.transpose(actions).astype(jnp.int32)
    out2 = _embed(actions_t, embedding_table)
    return out2[:, : _HIST * _HIDDEN].reshape(_BATCH, _HIST, _HIDDEN)


# speedup vs baseline: 1.1264x; 1.1264x over previous
import functools

import jax
import jax.numpy as jnp
from jax import lax
from jax.experimental import pallas as pl
from jax.experimental.pallas import tpu as pltpu
from jax.experimental.pallas import tpu_sc as plsc

_BATCH = 16384
_HIST = 50
_HIDDEN = 32
_B = _BATCH * _HIST
_NC, _NS = 2, 16
_NW = _NC * _NS
_BT = _BATCH // 128
_BT_PER_W = _BT // _NW
_G = 5
_NGRP = _HIST // _G

_mesh = plsc.VectorSubcoreMesh(core_axis_name="c", subcore_axis_name="s")


@functools.partial(
    pl.kernel,
    mesh=_mesh,
    out_type=jax.ShapeDtypeStruct((_BATCH, _HIST * _HIDDEN), jnp.float32),
    compiler_params=pltpu.CompilerParams(use_tc_tiling_on_sc=False),
    scratch_types=[
        pltpu.VMEM((_HIST, 128), jnp.int32),
        pltpu.VMEM((_G * 128, _HIDDEN), jnp.float32),
        pltpu.VMEM((_G * 128, _HIDDEN), jnp.float32),
        pltpu.VMEM((_G * 128, _HIDDEN), jnp.float32),
        pltpu.SemaphoreType.DMA,
        pltpu.SemaphoreType.DMA,
        pltpu.SemaphoreType.DMA,
        pltpu.SemaphoreType.DMA,
        pltpu.SemaphoreType.DMA,
        pltpu.SemaphoreType.DMA,
    ],
)
def _embed(actions_hbm, table_hbm, out_hbm, idx_slab, slot0, slot1, slot2,
           gs0, gs1, gs2, os0, os1, os2):
    wid = lax.axis_index("s") * _NC + lax.axis_index("c")
    slots = (slot0, slot1, slot2)
    gsems = (gs0, gs1, gs2)
    osems = (os0, os1, os2)

    def fire(g, btg, s):
        for j in range(_G):
            h = g * _G + j
            pltpu.async_copy(
                table_hbm.at[idx_slab.at[h]],
                slots[s].at[pl.ds(j * 128, 128)],
                gsems[s],
            )

    def drain_gathers(s):
        pltpu.make_async_copy(
            table_hbm.at[pl.ds(0, _G * 128)], slots[s], gsems[s]
        ).wait()

    def store(g, btg, s):
        for j in range(_G):
            h = g * _G + j
            pltpu.async_copy(
                slots[s].at[pl.ds(j * 128, 128)],
                out_hbm.at[pl.ds(btg * 128, 128), pl.ds(h * _HIDDEN, _HIDDEN)],
                osems[s],
            )

    def wait_store(s):
        pltpu.make_async_copy(
            slots[s],
            out_hbm.at[pl.ds(0, _G * 128), pl.ds(0, _HIDDEN)],
            osems[s],
        ).wait()

    def btile_body(bt, carry):
        btg = wid * _BT_PER_W + bt
        pltpu.sync_copy(
            actions_hbm.at[pl.ds(0, _HIST), pl.ds(btg * 128, 128)], idx_slab)

        fire(0, btg, 0)
        fire(1, btg, 1)
        for g in range(_NGRP):
            s = g % 3
            drain_gathers(s)
            if g + 2 < _NGRP:
                ns = (g + 2) % 3
                if g >= 1:
                    wait_store(ns)
                fire(g + 2, btg, ns)
            store(g, btg, s)
        wait_store(0)
        wait_store(1)
        wait_store(2)
        return carry

    lax.fori_loop(0, _BT_PER_W, btile_body, 0)


def kernel(actions, embedding_table):
    actions_t = jnp.transpose(actions).astype(jnp.int32)
    out2 = _embed(actions_t, embedding_table)
    return out2.reshape(_BATCH, _HIST, _HIDDEN)
